# Initial kernel scaffold; baseline (speedup 1.0000x reference)
import jax
import jax.numpy as jnp
from jax.experimental import pallas as pl


def kernel(in_feat, edge_index, root_emb0, root_emb1, ln_g0, ln_b0, ln_g1, ln_b1):
    raise NotImplementedError("write your pallas kernel here")


# speedup vs baseline: 17.8363x; 17.8363x over previous
import functools

import jax
import jax.numpy as jnp
from jax import lax
from jax.experimental import pallas as pl
from jax.experimental.pallas import tpu as pltpu
from jax.experimental.pallas import tpu_sc as plsc

N = 10000
E = 320000
D = 128
EPS = 1e-5

NC = 2
NS = 16
NW = NC * NS
EPW = E // NW
B = 80
NB = EPW // B
NBUF = 2
CH = 25
NCH = NB // CH
NPT = 640
NPAD = NPT * NS

_mesh = plsc.VectorSubcoreMesh(
    core_axis_name="c", subcore_axis_name="s", num_cores=NC, num_subcores=NS
)


LG = B // 16


@functools.partial(
    pl.kernel,
    out_type=jax.ShapeDtypeStruct((NC, NPAD), jnp.float32),
    mesh=_mesh,
    scratch_types=[
        pltpu.VMEM((CH, B), jnp.int32),
        pltpu.VMEM((B,), jnp.float32),
        pltpu.VMEM((NPT,), jnp.float32),
        pltpu.VMEM_SHARED((NPAD,), jnp.float32),
    ],
)
def _sc_counts(row_hbm, cnt_out, idx_v, ones_v, zbuf, cnt_sh):
    c = lax.axis_index("c")
    s = lax.axis_index("s")
    wid = c * NS + s
    zero16 = jnp.zeros((16,), jnp.float32)
    one16 = jnp.ones((16,), jnp.float32)
    for g in range(B // 16):
        ones_v[pl.ds(g * 16, 16)] = one16

    @pl.loop(0, NPT // 16)
    def _(i):
        zbuf[pl.ds(i * 16, 16)] = zero16

    pltpu.sync_copy(zbuf, cnt_sh.at[pl.ds(s * NPT, NPT)])
    plsc.subcore_barrier()

    for ch in range(NCH):
        pltpu.sync_copy(row_hbm.at[wid, ch], idx_v)

        @pl.loop(0, CH)
        def _(b):
            pltpu.sync_copy(ones_v, cnt_sh.at[idx_v.at[b]], add=True)

    plsc.subcore_barrier()
    pltpu.sync_copy(cnt_sh.at[pl.ds(s * NPT, NPT)], cnt_out.at[c, pl.ds(s * NPT, NPT)])


@functools.partial(
    pl.kernel,
    out_type=jax.ShapeDtypeStruct((NC, NPAD, D), jnp.float32),
    mesh=_mesh,
    scratch_types=(
        [
            pltpu.VMEM((CH, B), jnp.int32),
            pltpu.VMEM((CH, B), jnp.int32),
            pltpu.VMEM_SHARED((NPAD, D), jnp.float32),
        ]
        + [pltpu.VMEM((B, D), jnp.float32) for _ in range(NBUF)]
        + [pltpu.SemaphoreType.DMA for _ in range(2 * NBUF)]
    ),
)
def _sc_agg(z_hbm, row_hbm, col_hbm, zeros_hbm, t_out, ridx, cidx, t_sh, *rest):
    gbufs = rest[:NBUF]
    gsems = rest[NBUF : 2 * NBUF]
    ssems = rest[2 * NBUF :]
    c = lax.axis_index("c")
    s = lax.axis_index("s")
    wid = c * NS + s
    pltpu.sync_copy(zeros_hbm, t_sh.at[pl.ds(s * NPT, NPT)])
    plsc.subcore_barrier()

    def _process(b0, nbatch):
        gds = [
            pltpu.async_copy(z_hbm.at[ridx.at[b0 + j]], gbufs[j], gsems[j])
            for j in range(nbatch)
        ]
        sds = []
        for j in range(nbatch):
            gds[j].wait()
            sds.append(
                pltpu.async_copy(
                    gbufs[j], t_sh.at[cidx.at[b0 + j]], ssems[j], add=True
                )
            )
        for d in sds:
            d.wait()

    nmain = (CH // NBUF) * NBUF
    for ch in range(NCH):
        pltpu.sync_copy(row_hbm.at[wid, ch], ridx)
        pltpu.sync_copy(col_hbm.at[wid, ch], cidx)

        @pl.loop(0, nmain, step=NBUF)
        def _(b0):
            _process(b0, NBUF)

        for tail in range(nmain, CH):
            _process(tail, 1)

    plsc.subcore_barrier()
    pltpu.sync_copy(t_sh.at[pl.ds(s * NPT, NPT)], t_out.at[c, pl.ds(s * NPT, NPT)])


R = 1000
G = N // R


def _deg_terms(cnt_ref):
    deg = jnp.sum(cnt_ref[...], axis=1, keepdims=True) + 1.0
    return lax.rsqrt(deg), 1.0 / deg


def _tc_prep_body(cnt_ref, x_ref, re_ref, z_ref, r_ref):
    dis, invd = _deg_terms(cnt_ref)
    x = x_ref[...]
    z_ref[...] = dis * jnp.maximum(x, 0.0)
    r_ref[...] = jnp.maximum(x + re_ref[...], 0.0) * invd


def _tc_mid_body(t_ref, cnt_ref, r1_ref, re_ref, g_ref, b_ref, z_ref, r2_ref):
    dis, invd = _deg_terms(cnt_ref)
    pre = dis * (t_ref[0] + t_ref[1]) + r1_ref[...]
    m = jnp.mean(pre, axis=-1, keepdims=True)
    v = jnp.mean((pre - m) ** 2, axis=-1, keepdims=True)
    h = (pre - m) * lax.rsqrt(v + EPS) * g_ref[...] + b_ref[...]
    h = jnp.maximum(h, 0.0)
    z_ref[...] = dis * h
    r2_ref[...] = jnp.maximum(h + re_ref[...], 0.0) * invd


def _tc_final_body(t_ref, cnt_ref, r2_ref, g_ref, b_ref, o_ref):
    dis, _ = _deg_terms(cnt_ref)
    pre = dis * (t_ref[0] + t_ref[1]) + r2_ref[...]
    m = jnp.mean(pre, axis=-1, keepdims=True)
    v = jnp.mean((pre - m) ** 2, axis=-1, keepdims=True)
    o_ref[...] = (pre - m) * lax.rsqrt(v + EPS) * g_ref[...] + b_ref[...]


_cnt_spec = pl.BlockSpec((R, NC), lambda i: (i, 0))
_t_spec = pl.BlockSpec((NC, R, D), lambda i: (0, i, 0))
_nd_spec = pl.BlockSpec((R, D), lambda i: (i, 0))
_vec_spec = pl.BlockSpec((1, D), lambda i: (0, 0))
_nd_shape = jax.ShapeDtypeStruct((N, D), jnp.float32)


def _tc_prep(cnt, x, re0):
    return pl.pallas_call(
        _tc_prep_body,
        grid=(G,),
        in_specs=[_cnt_spec, _nd_spec, _vec_spec],
        out_specs=[_nd_spec, _nd_spec],
        out_shape=[_nd_shape, _nd_shape],
    )(cnt, x, re0)


def _tc_mid(t, cnt, r1, re1, g0, b0):
    return pl.pallas_call(
        _tc_mid_body,
        grid=(G,),
        in_specs=[_t_spec, _cnt_spec, _nd_spec, _vec_spec, _vec_spec, _vec_spec],
        out_specs=[_nd_spec, _nd_spec],
        out_shape=[_nd_shape, _nd_shape],
    )(t, cnt, r1, re1, g0, b0)


def _tc_final(t, cnt, r2, g1, b1):
    return pl.pallas_call(
        _tc_final_body,
        grid=(G,),
        in_specs=[_t_spec, _cnt_spec, _nd_spec, _vec_spec, _vec_spec],
        out_specs=_nd_spec,
        out_shape=_nd_shape,
    )(t, cnt, r2, g1, b1)


def kernel(in_feat, edge_index, root_emb0, root_emb1, ln_g0, ln_b0, ln_g1, ln_b1):
    row = edge_index[0].astype(jnp.int32).reshape(NW, NCH, CH, B)
    col = edge_index[1].astype(jnp.int32).reshape(NW, NCH, CH, B)
    zerosD = jnp.zeros((NPT, D), jnp.float32)

    cnt = _sc_counts(row).T
    z1, r1 = _tc_prep(cnt, in_feat, root_emb0.reshape(1, D))
    t1 = _sc_agg(z1, row, col, zerosD)
    z2, r2 = _tc_mid(t1, cnt, r1, root_emb1.reshape(1, D),
                     ln_g0.reshape(1, D), ln_b0.reshape(1, D))
    t2 = _sc_agg(z2, row, col, zerosD)
    return _tc_final(t2, cnt, r2, ln_g1.reshape(1, D), ln_b1.reshape(1, D))
